# Initial kernel scaffold; baseline (speedup 1.0000x reference)
import jax
import jax.numpy as jnp
from jax.experimental import pallas as pl


def kernel(edge_index, num_nodes, spd_bias_weight):
    raise NotImplementedError("write your pallas kernel here")


# speedup vs baseline: 1.2745x; 1.2745x over previous
import functools

import jax
import jax.numpy as jnp
from jax import lax
from jax.experimental import pallas as pl
from jax.experimental.pallas import tpu as pltpu
from jax.experimental.pallas import tpu_sc as plsc

N = 2048
E = 32768
H = 8
MAX_D = 10
TBL = MAX_D + 2

_NC = 2
_NS = 16


@functools.cache
def _sc_mesh():
    return plsc.VectorSubcoreMesh(
        core_axis_name="c", subcore_axis_name="s",
        num_cores=_NC, num_subcores=_NS)


_ZROWS = N // (_NC * _NS)
_ZCHUNK = 16384
_ZITER = _ZROWS * N // _ZCHUNK
_EPT = E // _NS


def _adj_body(edge_hbm, adj_hbm, zbuf, ones_v, idx1, idx2, srcv, dstv, sem1, sem2):
    c = lax.axis_index("c")
    s = lax.axis_index("s")
    half0 = c * (N // _NC)
    dummy = half0 * N + half0

    def zb(i, _):
        zbuf[pl.ds(i * 16, 16)] = jnp.zeros((16,), jnp.float32)
        return 0
    lax.fori_loop(0, _ZCHUNK // 16, zb, 0)
    for g in range(8):
        ones_v[pl.ds(g * 16, 16)] = jnp.ones((16,), jnp.float32)

    zbase = (half0 + s * _ZROWS) * N

    def zdma(i, _):
        off = pl.multiple_of(zbase + i * _ZCHUNK, _ZCHUNK)
        pltpu.sync_copy(zbuf, adj_hbm.at[pl.ds(off, _ZCHUNK)])
        return 0
    lax.fori_loop(0, _ZITER, zdma, 0)

    plsc.subcore_barrier()

    eoff = pl.multiple_of(s * _EPT, _EPT)
    pltpu.sync_copy(edge_hbm.at[0, pl.ds(eoff, _EPT)], srcv)
    pltpu.sync_copy(edge_hbm.at[1, pl.ds(eoff, _EPT)], dstv)

    def chunk(k, _):
        for g in range(8):
            off = k * 128 + g * 16
            sv = srcv[pl.ds(off, 16)]
            dv = dstv[pl.ds(off, 16)]
            p1 = sv * N + dv
            p2 = dv * N + sv
            ok1 = (sv >= half0) & (sv < half0 + N // _NC)
            ok2 = (dv >= half0) & (dv < half0 + N // _NC)
            idx1[pl.ds(g * 16, 16)] = jnp.where(ok1, p1, dummy)
            idx2[pl.ds(g * 16, 16)] = jnp.where(ok2, p2, dummy)
        d1 = pltpu.async_copy(ones_v, adj_hbm.at[idx1], sem1)
        d2 = pltpu.async_copy(ones_v, adj_hbm.at[idx2], sem2)
        d1.wait()
        d2.wait()
        return 0
    lax.fori_loop(0, _EPT // 128, chunk, 0)


@functools.cache
def _adj_scatter():
  return pl.kernel(
    _adj_body,
    out_type=jax.ShapeDtypeStruct((N * N,), jnp.float32),
    mesh=_sc_mesh(),
    scratch_types=[
        pltpu.VMEM((_ZCHUNK,), jnp.float32),
        pltpu.VMEM((128,), jnp.float32),
        pltpu.VMEM((128,), jnp.int32),
        pltpu.VMEM((128,), jnp.int32),
        pltpu.VMEM((_EPT,), jnp.int32),
        pltpu.VMEM((_EPT,), jnp.int32),
        pltpu.SemaphoreType.DMA,
        pltpu.SemaphoreType.DMA,
    ],
  )


_RB = 256


def _cast_body(adj_ref, out_ref):
    out_ref[...] = adj_ref[...].astype(jnp.bfloat16)


def _bfs_body(adj_ref, idxp_ref):
    i = pl.program_id(0)
    row0 = i * _RB
    adjb = adj_ref[...]

    rows = lax.broadcasted_iota(jnp.int32, (_RB, N), 0) + row0
    cols = lax.broadcasted_iota(jnp.int32, (_RB, N), 1)
    reach0 = (rows == cols).astype(jnp.bfloat16)

    def cond(carry):
        d, done, _, _, _ = carry
        return jnp.logical_and(d <= MAX_D - 1, jnp.logical_not(done))

    def body(carry):
        d, _, prevcnt, reach, s_acc = carry
        nxt = jnp.dot(reach, adjb, preferred_element_type=jnp.float32) > 0.0
        newr = jnp.maximum(reach, nxt.astype(jnp.bfloat16))
        newr_f = newr.astype(jnp.float32)
        cnt = jnp.sum(newr_f)
        conv = cnt == prevcnt
        extra = jnp.where(conv, (MAX_D - 1 - d).astype(jnp.float32), 0.0)
        s_acc = s_acc + newr_f * (1.0 + extra)
        return d + 1, conv, cnt, newr, s_acc

    init = (jnp.int32(1), jnp.bool_(False), jnp.float32(_RB),
            reach0, reach0.astype(jnp.float32))
    _, _, _, _, s_acc = lax.while_loop(cond, body, init)

    idx = (MAX_D + 1.0) - s_acc
    m = jnp.where(cols % 2 == 0, jnp.float32(TBL), jnp.float32(1.0))
    t = (idx * m).astype(jnp.bfloat16)
    pr = lax.broadcasted_iota(jnp.int32, (N, N // 2), 0)
    pc = lax.broadcasted_iota(jnp.int32, (N, N // 2), 1)
    psel = (pr // 2 == pc).astype(jnp.bfloat16)
    tp = jnp.dot(t, psel, preferred_element_type=jnp.float32)
    idxp_ref[...] = tp.astype(jnp.int32)


_cast = pl.pallas_call(
    _cast_body,
    grid=(N // _RB,),
    in_specs=[pl.BlockSpec((_RB, N), lambda i: (i, 0))],
    out_specs=pl.BlockSpec((_RB, N), lambda i: (i, 0)),
    out_shape=jax.ShapeDtypeStruct((N, N), jnp.bfloat16),
)

_bfs = pl.pallas_call(
    _bfs_body,
    grid=(N // _RB,),
    in_specs=[pl.BlockSpec((N, N), lambda i: (0, 0))],
    out_specs=pl.BlockSpec((_RB, N // 2), lambda i: (i, 0)),
    out_shape=jax.ShapeDtypeStruct((N, N // 2), jnp.int32),
)


_ROWS_TOTAL = N * N // 2
_RPT = _ROWS_TOTAL // (_NC * _NS)
_CHUNK = 2048
_NCHUNK = _RPT // _CHUNK
_SUB = 16


def _gather_body(idxp_hbm, tbl_hbm, out_hbm, idxc, rowsb, sem):
    c = lax.axis_index("c")
    s = lax.axis_index("s")
    wid = s * _NC + c
    base = wid * _RPT

    def chunk(g, _):
        r0 = pl.multiple_of(base + g * _CHUNK, _CHUNK)
        pltpu.sync_copy(idxp_hbm.at[pl.ds(pl.multiple_of(r0 // 128, 16), _CHUNK // 128)], idxc)
        copies = []
        for j in range(_SUB):
            copies.append(pltpu.async_copy(
                tbl_hbm.at[idxc.at[j]],
                rowsb.at[pl.ds(j * 128, 128)],
                sem,
            ))
        for d in copies:
            d.wait()
        pltpu.sync_copy(rowsb, out_hbm.at[pl.ds(pl.multiple_of(r0, _CHUNK), _CHUNK)])
        return 0
    lax.fori_loop(0, _NCHUNK, chunk, 0)


@functools.cache
def _pair_gather():
  return pl.kernel(
    _gather_body,
    out_type=jax.ShapeDtypeStruct((_ROWS_TOTAL, 16), jnp.float32),
    mesh=_sc_mesh(),
    compiler_params=pltpu.CompilerParams(use_tc_tiling_on_sc=False),
    scratch_types=[
        pltpu.VMEM((_CHUNK // 128, 128), jnp.int32),
        pltpu.VMEM((_CHUNK, 16), jnp.float32),
        pltpu.SemaphoreType.DMA,
    ],
  )


def kernel(edge_index, num_nodes, spd_bias_weight):
    del num_nodes
    edge_index = edge_index.astype(jnp.int32)
    w = spd_bias_weight.astype(jnp.float32)

    adj = _adj_scatter()(edge_index)
    adjb = _cast(adj.reshape(N, N))
    idxp = _bfs(adjb)
    tbl2 = jnp.concatenate(
        [jnp.repeat(w, TBL, axis=0), jnp.tile(w, (TBL, 1))], axis=1)
    out = _pair_gather()(idxp.reshape(_ROWS_TOTAL // 128, 128), tbl2)
    return out.reshape(N, N, H)


# speedup vs baseline: 6.5252x; 5.1198x over previous
import functools

import jax
import jax.numpy as jnp
from jax import lax
from jax.experimental import pallas as pl
from jax.experimental.pallas import tpu as pltpu
from jax.experimental.pallas import tpu_sc as plsc

N = 2048
E = 32768
H = 8
MAX_D = 10
TBL = MAX_D + 2

_NC = 2
_NS = 16
_NT = _NC * _NS


@functools.cache
def _sc_mesh():
    return plsc.VectorSubcoreMesh(
        core_axis_name="c", subcore_axis_name="s",
        num_cores=_NC, num_subcores=_NS)


_WROWS = 32
_PASS = N // (_NT * _WROWS)
_ECHUNK = 8192


def _adj_body(edge_hbm, adj_hbm, win, srcv, dstv):
    c = lax.axis_index("c")
    s = lax.axis_index("s")
    tid = c * _NS + s
    ones16 = jnp.ones((16,), jnp.float32)

    for p in range(_PASS):
        rowbase = tid * (_WROWS * _PASS) + p * _WROWS

        def zro(i, _):
            win[pl.ds(i * 16, 16)] = jnp.zeros((16,), jnp.float32)
            return 0
        lax.fori_loop(0, _WROWS * N // 16, zro, 0)

        for ec in range(E // _ECHUNK):
            pltpu.sync_copy(edge_hbm.at[0, pl.ds(ec * _ECHUNK, _ECHUNK)], srcv)
            pltpu.sync_copy(edge_hbm.at[1, pl.ds(ec * _ECHUNK, _ECHUNK)], dstv)

            def grp(t, _):
                sv = srcv[pl.ds(t * 16, 16)]
                dv = dstv[pl.ds(t * 16, 16)]
                for a, b in ((sv, dv), (dv, sv)):
                    r = a - rowbase
                    ok = (r >= 0) & (r < _WROWS)
                    li = jnp.where(ok, r * N + b, 0)
                    plsc.store_scatter(win, [li], ones16, mask=ok)
                return 0
            lax.fori_loop(0, _ECHUNK // 16, grp, 0)

        off = pl.multiple_of(rowbase * N, _WROWS * N)
        pltpu.sync_copy(win, adj_hbm.at[pl.ds(off, _WROWS * N)])


@functools.cache
def _adj_scatter():
  return pl.kernel(
    _adj_body,
    out_type=jax.ShapeDtypeStruct((N * N,), jnp.float32),
    mesh=_sc_mesh(),
    compiler_params=pltpu.CompilerParams(use_tc_tiling_on_sc=False, needs_layout_passes=False),
    scratch_types=[
        pltpu.VMEM((_WROWS * N,), jnp.float32),
        pltpu.VMEM((_ECHUNK,), jnp.int32),
        pltpu.VMEM((_ECHUNK,), jnp.int32),
    ],
  )


_RB = 256


def _bfs_body(adj_ref, idxp_ref):
    i = pl.program_id(0)
    row0 = i * _RB
    adjb = adj_ref[...].astype(jnp.bfloat16)

    rows = lax.broadcasted_iota(jnp.int32, (_RB, N), 0) + row0
    cols = lax.broadcasted_iota(jnp.int32, (_RB, N), 1)
    reach0 = (rows == cols).astype(jnp.bfloat16)

    def cond(carry):
        d, done, _, _, _ = carry
        return jnp.logical_and(d <= MAX_D - 1, jnp.logical_not(done))

    def body(carry):
        d, _, prevcnt, reach, s_acc = carry
        nxt = jnp.dot(reach, adjb, preferred_element_type=jnp.float32) > 0.0
        newr = jnp.maximum(reach, nxt.astype(jnp.bfloat16))
        newr_f = newr.astype(jnp.float32)
        cnt = jnp.sum(newr_f)
        conv = cnt == prevcnt
        extra = jnp.where(conv, (MAX_D - 1 - d).astype(jnp.float32), 0.0)
        s_acc = s_acc + newr_f * (1.0 + extra)
        return d + 1, conv, cnt, newr, s_acc

    init = (jnp.int32(1), jnp.bool_(False), jnp.float32(_RB),
            reach0, reach0.astype(jnp.float32))
    _, _, _, _, s_acc = lax.while_loop(cond, body, init)

    idx = (MAX_D + 1.0) - s_acc
    m = jnp.where(cols % 2 == 0, jnp.float32(TBL), jnp.float32(1.0))
    t = (idx * m).astype(jnp.bfloat16)
    pr = lax.broadcasted_iota(jnp.int32, (N, N // 2), 0)
    pc = lax.broadcasted_iota(jnp.int32, (N, N // 2), 1)
    psel = (pr // 2 == pc).astype(jnp.bfloat16)
    tp = jnp.dot(t, psel, preferred_element_type=jnp.float32)
    idxp_ref[...] = tp.astype(jnp.int32)


@functools.cache
def _bfs():
  return pl.pallas_call(
    _bfs_body,
    grid=(N // _RB,),
    in_specs=[pl.BlockSpec((N, N), lambda i: (0, 0))],
    out_specs=pl.BlockSpec((_RB, N // 2), lambda i: (i, 0)),
    out_shape=jax.ShapeDtypeStruct((N, N // 2), jnp.int32),
  )


_NPAIR = N * N // 2
_PPT = _NPAIR // _NT
_CHUNK = 2048
_NCHUNK = _PPT // _CHUNK


def _gather_body(idxp_hbm, tbl_hbm, out_hbm, tblv, ib0, ib1, rb0, rb1,
                 semi0, semi1, semo0, semo1):
    c = lax.axis_index("c")
    s = lax.axis_index("s")
    tid = c * _NS + s
    pbase = tid * _PPT

    pltpu.sync_copy(tbl_hbm, tblv)

    ibs = (ib0, ib1)
    rbs = (rb0, rb1)
    semis = (semi0, semi1)
    semos = (semo0, semo1)

    iotam = lax.iota(jnp.int32, 16) * 16

    def idx_off(g):
        return pl.multiple_of(pbase + g * _CHUNK, _CHUNK)

    def out_off(g):
        return pl.multiple_of((pbase + g * _CHUNK) * 16, _CHUNK * 16)

    idma = [None, None]
    odma = [None, None]
    idma[0] = pltpu.async_copy(
        idxp_hbm.at[pl.ds(idx_off(0), _CHUNK)], ibs[0], semis[0])

    for g in range(_NCHUNK):
        cur = g % 2
        nxt = (g + 1) % 2
        if g + 1 < _NCHUNK:
            idma[nxt] = pltpu.async_copy(
                idxp_hbm.at[pl.ds(idx_off(g + 1), _CHUNK)], ibs[nxt], semis[nxt])
        idma[cur].wait()
        if odma[cur] is not None:
            odma[cur].wait()

        ib = ibs[cur]
        rb = rbs[cur]

        def block(b, _):
            pib = ib[pl.ds(b * 16, 16)]
            pb = pib * 16
            ob = iotam + b * 256
            for col in range(16):
                v = plsc.load_gather(tblv, [pb + col])
                plsc.store_scatter(rb, [ob + col], v)
            return 0
        lax.fori_loop(0, _CHUNK // 16, block, 0)

        odma[cur] = pltpu.async_copy(
            rb, out_hbm.at[pl.ds(out_off(g), _CHUNK * 16)], semos[cur])

    odma[0].wait()
    odma[1].wait()


@functools.cache
def _pair_gather():
  return pl.kernel(
    _gather_body,
    out_type=jax.ShapeDtypeStruct((N * N * H,), jnp.float32),
    mesh=_sc_mesh(),
    compiler_params=pltpu.CompilerParams(use_tc_tiling_on_sc=False, needs_layout_passes=False),
    scratch_types=[
        pltpu.VMEM((TBL * TBL * 16,), jnp.float32),
        pltpu.VMEM((_CHUNK,), jnp.int32),
        pltpu.VMEM((_CHUNK,), jnp.int32),
        pltpu.VMEM((_CHUNK * 16,), jnp.float32),
        pltpu.VMEM((_CHUNK * 16,), jnp.float32),
        pltpu.SemaphoreType.DMA,
        pltpu.SemaphoreType.DMA,
        pltpu.SemaphoreType.DMA,
        pltpu.SemaphoreType.DMA,
    ],
  )


def kernel(edge_index, num_nodes, spd_bias_weight):
    del num_nodes
    edge_index = edge_index.astype(jnp.int32)
    w = spd_bias_weight.astype(jnp.float32)

    adj = _adj_scatter()(edge_index)
    idxp = _bfs()(adj.reshape(N, N))
    tbl2 = jnp.concatenate(
        [jnp.repeat(w, TBL, axis=0), jnp.tile(w, (TBL, 1))], axis=1)
    out = _pair_gather()(idxp.reshape(-1), tbl2.reshape(-1))
    return out.reshape(N, N, H)


# speedup vs baseline: 6.5484x; 1.0036x over previous
import functools

import jax
import jax.numpy as jnp
from jax import lax
from jax.experimental import pallas as pl
from jax.experimental.pallas import tpu as pltpu
from jax.experimental.pallas import tpu_sc as plsc

N = 2048
E = 32768
H = 8
MAX_D = 10
TBL = MAX_D + 2

_NC = 2
_NS = 16
_NT = _NC * _NS


@functools.cache
def _sc_mesh():
    return plsc.VectorSubcoreMesh(
        core_axis_name="c", subcore_axis_name="s",
        num_cores=_NC, num_subcores=_NS)


_WROWS = 32
_PASS = N // (_NT * _WROWS)
_ECHUNK = 8192


def _adj_body(edge_hbm, adj_hbm, win, srcv, dstv):
    c = lax.axis_index("c")
    s = lax.axis_index("s")
    tid = c * _NS + s
    ones16 = jnp.ones((16,), jnp.float32)

    for p in range(_PASS):
        rowbase = tid * (_WROWS * _PASS) + p * _WROWS

        def zro(i, _):
            win[pl.ds(i * 16, 16)] = jnp.zeros((16,), jnp.float32)
            return 0
        lax.fori_loop(0, _WROWS * N // 16, zro, 0)

        for ec in range(E // _ECHUNK):
            pltpu.sync_copy(edge_hbm.at[0, pl.ds(ec * _ECHUNK, _ECHUNK)], srcv)
            pltpu.sync_copy(edge_hbm.at[1, pl.ds(ec * _ECHUNK, _ECHUNK)], dstv)

            def grp(t, _):
                sv = srcv[pl.ds(t * 16, 16)]
                dv = dstv[pl.ds(t * 16, 16)]
                for a, b in ((sv, dv), (dv, sv)):
                    r = a - rowbase
                    ok = (r >= 0) & (r < _WROWS)
                    li = jnp.where(ok, r * N + b, 0)
                    plsc.store_scatter(win, [li], ones16, mask=ok)
                return 0
            lax.fori_loop(0, _ECHUNK // 16, grp, 0)

        off = pl.multiple_of(rowbase * N, _WROWS * N)
        pltpu.sync_copy(win, adj_hbm.at[pl.ds(off, _WROWS * N)])


@functools.cache
def _adj_scatter():
  return pl.kernel(
    _adj_body,
    out_type=jax.ShapeDtypeStruct((N * N,), jnp.float32),
    mesh=_sc_mesh(),
    compiler_params=pltpu.CompilerParams(use_tc_tiling_on_sc=False, needs_layout_passes=False),
    scratch_types=[
        pltpu.VMEM((_WROWS * N,), jnp.float32),
        pltpu.VMEM((_ECHUNK,), jnp.int32),
        pltpu.VMEM((_ECHUNK,), jnp.int32),
    ],
  )


_RB = 256


def _bfs_body(adj_ref, idxp_ref):
    i = pl.program_id(0)
    row0 = i * _RB
    adjb = adj_ref[...].reshape(N, N).astype(jnp.bfloat16)

    rows = lax.broadcasted_iota(jnp.int32, (_RB, N), 0) + row0
    cols = lax.broadcasted_iota(jnp.int32, (_RB, N), 1)
    reach0 = (rows == cols).astype(jnp.bfloat16)

    def cond(carry):
        d, done, _, _, _ = carry
        return jnp.logical_and(d <= MAX_D - 1, jnp.logical_not(done))

    def body(carry):
        d, _, prevcnt, reach, s_acc = carry
        nxt = jnp.dot(reach, adjb, preferred_element_type=jnp.float32) > 0.0
        newr = jnp.maximum(reach, nxt.astype(jnp.bfloat16))
        newr_f = newr.astype(jnp.float32)
        cnt = jnp.sum(newr_f)
        conv = cnt == prevcnt
        extra = jnp.where(conv, (MAX_D - 1 - d).astype(jnp.float32), 0.0)
        s_acc = s_acc + newr_f * (1.0 + extra)
        return d + 1, conv, cnt, newr, s_acc

    init = (jnp.int32(1), jnp.bool_(False), jnp.float32(_RB),
            reach0, reach0.astype(jnp.float32))
    _, _, _, _, s_acc = lax.while_loop(cond, body, init)

    idx = (MAX_D + 1.0) - s_acc
    m = jnp.where(cols % 2 == 0, jnp.float32(TBL), jnp.float32(1.0))
    t = (idx * m).astype(jnp.bfloat16)
    pr = lax.broadcasted_iota(jnp.int32, (N, N // 2), 0)
    pc = lax.broadcasted_iota(jnp.int32, (N, N // 2), 1)
    psel = (pr // 2 == pc).astype(jnp.bfloat16)
    tp = jnp.dot(t, psel, preferred_element_type=jnp.float32)
    idxp_ref[...] = tp.astype(jnp.int32).reshape(_RB * 8, 128)


@functools.cache
def _bfs():
  return pl.pallas_call(
    _bfs_body,
    grid=(N // _RB,),
    in_specs=[pl.BlockSpec((N * 16, 128), lambda i: (0, 0))],
    out_specs=pl.BlockSpec((_RB * 8, 128), lambda i: (i, 0)),
    out_shape=jax.ShapeDtypeStruct((N * 8, 128), jnp.int32),
  )


_NPAIR = N * N // 2
_PPT = _NPAIR // _NT
_CHUNK = 2048
_NCHUNK = _PPT // _CHUNK


_IPC = _CHUNK * 2 // N


def _gather_body(idxp_hbm, tbl_hbm, out_hbm, tblv, ib0, ib1, rb0, rb1,
                 semi0, semi1, semo0, semo1):
    c = lax.axis_index("c")
    s = lax.axis_index("s")
    tid = c * _NS + s
    pbase = tid * _PPT
    ibase = tid * (_NCHUNK * _IPC)

    pltpu.sync_copy(tbl_hbm, tblv)

    ibs = (ib0, ib1)
    rbs = (rb0, rb1)
    semis = (semi0, semi1)
    semos = (semo0, semo1)

    iota2 = lax.iota(jnp.int32, 16) * 2
    hsplats = [jnp.full((16,), col & 7, jnp.int32) for col in range(16)]

    def idx_off(g):
        return pl.multiple_of(pbase + g * _CHUNK, _CHUNK)

    idma = [None, None]
    odma = [None, None]
    idma[0] = pltpu.async_copy(
        idxp_hbm.at[pl.ds(idx_off(0), _CHUNK)], ibs[0], semis[0])

    for g in range(_NCHUNK):
        cur = g % 2
        nxt = (g + 1) % 2
        if g + 1 < _NCHUNK:
            idma[nxt] = pltpu.async_copy(
                idxp_hbm.at[pl.ds(idx_off(g + 1), _CHUNK)], ibs[nxt], semis[nxt])
        idma[cur].wait()
        if odma[cur] is not None:
            odma[cur].wait()

        ib = ibs[cur]
        rb = rbs[cur]

        def block(b, _):
            pib = ib[pl.ds(b * 16, 16)]
            pb = pib * 16
            isp = jnp.full((16,), b >> 6, jnp.int32)
            jlo = iota2 + (b & 63) * 32
            jhi = jlo + 1
            for col in range(16):
                v = plsc.load_gather(tblv, [pb + col])
                jv = jlo if col < 8 else jhi
                plsc.store_scatter(rb, [isp, jv, hsplats[col]], v)
            return 0
        lax.fori_loop(0, _CHUNK // 16, block, 0)

        odma[cur] = pltpu.async_copy(
            rb, out_hbm.at[pl.ds(pl.multiple_of(ibase + g * _IPC, _IPC), _IPC)],
            semos[cur])

    odma[0].wait()
    odma[1].wait()


@functools.cache
def _pair_gather():
  return pl.kernel(
    _gather_body,
    out_type=jax.ShapeDtypeStruct((N, N, H), jnp.float32),
    mesh=_sc_mesh(),
    compiler_params=pltpu.CompilerParams(use_tc_tiling_on_sc=False, needs_layout_passes=False),
    scratch_types=[
        pltpu.VMEM((TBL * TBL * 16,), jnp.float32),
        pltpu.VMEM((_CHUNK,), jnp.int32),
        pltpu.VMEM((_CHUNK,), jnp.int32),
        pltpu.VMEM((_IPC, N, H), jnp.float32),
        pltpu.VMEM((_IPC, N, H), jnp.float32),
        pltpu.SemaphoreType.DMA,
        pltpu.SemaphoreType.DMA,
        pltpu.SemaphoreType.DMA,
        pltpu.SemaphoreType.DMA,
    ],
  )


def kernel(edge_index, num_nodes, spd_bias_weight):
    del num_nodes
    edge_index = edge_index.astype(jnp.int32)
    w = spd_bias_weight.astype(jnp.float32)

    adj = _adj_scatter()(edge_index)
    idxp = _bfs()(adj.reshape(N * 16, 128))
    tbl2 = jnp.concatenate(
        [jnp.repeat(w, TBL, axis=0), jnp.tile(w, (TBL, 1))], axis=1)
    return _pair_gather()(idxp.reshape(-1), tbl2.reshape(-1))


# speedup vs baseline: 26.1657x; 3.9958x over previous
import functools

import jax
import jax.numpy as jnp
from jax import lax
from jax.experimental import pallas as pl
from jax.experimental.pallas import tpu as pltpu
from jax.experimental.pallas import tpu_sc as plsc

N = 2048
E = 32768
H = 8
MAX_D = 10
TBL = MAX_D + 2

_NC = 2
_NS = 16
_NT = _NC * _NS


@functools.cache
def _sc_mesh():
    return plsc.VectorSubcoreMesh(
        core_axis_name="c", subcore_axis_name="s",
        num_cores=_NC, num_subcores=_NS)


_WROWS = 32
_PASS = N // (_NT * _WROWS)
_ECHUNK = 8192


def _adj_body(edge_hbm, adj_hbm, win, srcv, dstv):
    c = lax.axis_index("c")
    s = lax.axis_index("s")
    tid = c * _NS + s
    ones16 = jnp.ones((16,), jnp.float32)

    for p in range(_PASS):
        rowbase = tid * (_WROWS * _PASS) + p * _WROWS

        def zro(i, _):
            win[pl.ds(i * 16, 16)] = jnp.zeros((16,), jnp.float32)
            return 0
        lax.fori_loop(0, _WROWS * N // 16, zro, 0)

        for ec in range(E // _ECHUNK):
            pltpu.sync_copy(edge_hbm.at[0, pl.ds(ec * _ECHUNK, _ECHUNK)], srcv)
            pltpu.sync_copy(edge_hbm.at[1, pl.ds(ec * _ECHUNK, _ECHUNK)], dstv)

            def grp(t, _):
                sv = srcv[pl.ds(t * 16, 16)]
                dv = dstv[pl.ds(t * 16, 16)]
                for a, b in ((sv, dv), (dv, sv)):
                    r = a - rowbase
                    ok = (r >= 0) & (r < _WROWS)
                    li = jnp.where(ok, r * N + b, 0)
                    plsc.store_scatter(win, [li], ones16, mask=ok)
                return 0
            lax.fori_loop(0, _ECHUNK // 16, grp, 0)

        off = pl.multiple_of(rowbase * N, _WROWS * N)
        pltpu.sync_copy(win, adj_hbm.at[pl.ds(off, _WROWS * N)])


@functools.cache
def _adj_scatter():
  return pl.kernel(
    _adj_body,
    out_type=jax.ShapeDtypeStruct((N * N,), jnp.float32),
    mesh=_sc_mesh(),
    compiler_params=pltpu.CompilerParams(use_tc_tiling_on_sc=False, needs_layout_passes=False),
    scratch_types=[
        pltpu.VMEM((_WROWS * N,), jnp.float32),
        pltpu.VMEM((_ECHUNK,), jnp.int32),
        pltpu.VMEM((_ECHUNK,), jnp.int32),
    ],
  )


_RB = 256


def _bfs_body(adj_ref, idxp_ref):
    i = pl.program_id(0)
    row0 = i * _RB
    adjb = adj_ref[...].reshape(N, N).astype(jnp.bfloat16)

    rows = lax.broadcasted_iota(jnp.int32, (_RB, N), 0) + row0
    cols = lax.broadcasted_iota(jnp.int32, (_RB, N), 1)
    reach0 = (rows == cols).astype(jnp.bfloat16)

    def cond(carry):
        d, done, _, _, _ = carry
        return jnp.logical_and(d <= MAX_D - 1, jnp.logical_not(done))

    def body(carry):
        d, _, prevcnt, reach, s_acc = carry
        nxt = jnp.dot(reach, adjb, preferred_element_type=jnp.float32) > 0.0
        newr = jnp.maximum(reach, nxt.astype(jnp.bfloat16))
        newr_f = newr.astype(jnp.float32)
        cnt = jnp.sum(newr_f)
        conv = cnt == prevcnt
        extra = jnp.where(conv, (MAX_D - 1 - d).astype(jnp.float32), 0.0)
        s_acc = s_acc + newr_f * (1.0 + extra)
        return d + 1, conv, cnt, newr, s_acc

    init = (jnp.int32(1), jnp.bool_(False), jnp.float32(_RB),
            reach0, reach0.astype(jnp.float32))
    _, _, _, _, s_acc = lax.while_loop(cond, body, init)

    idx = (MAX_D + 1.0) - s_acc
    idxp_ref[...] = idx.astype(jnp.int32).reshape(_RB * 16, 128)


@functools.cache
def _bfs():
  return pl.pallas_call(
    _bfs_body,
    grid=(N // _RB,),
    in_specs=[pl.BlockSpec((N * 16, 128), lambda i: (0, 0))],
    out_specs=pl.BlockSpec((_RB * 16, 128), lambda i: (i, 0)),
    out_shape=jax.ShapeDtypeStruct((N * 16, 128), jnp.int32),
  )


_IPT = N // _NT
_ICH = 2
_NCHUNK = _IPT // _ICH
_CIDX = _ICH * N
_CROW = _ICH * 16 * H


def _gather_body(idx_hbm, w_hbm, out_hbm, tblv, ib0, ib1, rb0, rb1,
                 semi0, semi1, semo0, semo1):
    c = lax.axis_index("c")
    s = lax.axis_index("s")
    tid = c * _NS + s
    pbase = tid * (_NCHUNK * _CIDX)
    obase = tid * (_NCHUNK * _CROW)

    pltpu.sync_copy(w_hbm, tblv)

    ibs = (ib0, ib1)
    rbs = (rb0, rb1)
    semis = (semi0, semi1)
    semos = (semo0, semo1)

    def idx_off(g):
        return pl.multiple_of(pbase + g * _CIDX, _CIDX)

    idma = [None, None]
    odma = [None, None]
    idma[0] = pltpu.async_copy(
        idx_hbm.at[pl.ds(idx_off(0), _CIDX)], ibs[0], semis[0])

    for g in range(_NCHUNK):
        cur = g % 2
        nxt = (g + 1) % 2
        if g + 1 < _NCHUNK:
            idma[nxt] = pltpu.async_copy(
                idx_hbm.at[pl.ds(idx_off(g + 1), _CIDX)], ibs[nxt], semis[nxt])
        idma[cur].wait()
        if odma[cur] is not None:
            odma[cur].wait()

        ib = ibs[cur]
        rb = rbs[cur]

        def block(b, _):
            iv = ib[pl.ds(b * 16, 16)]
            a0 = iv * H
            row0 = (b >> 7) * (16 * H) + ((b >> 3) & 15) * H
            lane0 = (b & 7) * 16
            for h in range(H):
                v = plsc.load_gather(tblv, [a0 + h])
                rb[row0 + h, pl.ds(lane0, 16)] = v
            return 0
        lax.fori_loop(0, _CIDX // 16, block, 0)

        odma[cur] = pltpu.async_copy(
            rb, out_hbm.at[pl.ds(pl.multiple_of(obase + g * _CROW, _CROW), _CROW)],
            semos[cur])

    odma[0].wait()
    odma[1].wait()


@functools.cache
def _hgather():
  return pl.kernel(
    _gather_body,
    out_type=jax.ShapeDtypeStruct((N * 16 * H, 128), jnp.float32),
    mesh=_sc_mesh(),
    compiler_params=pltpu.CompilerParams(use_tc_tiling_on_sc=False, needs_layout_passes=False),
    scratch_types=[
        pltpu.VMEM((TBL * H,), jnp.float32),
        pltpu.VMEM((_CIDX,), jnp.int32),
        pltpu.VMEM((_CIDX,), jnp.int32),
        pltpu.VMEM((_CROW, 128), jnp.float32),
        pltpu.VMEM((_CROW, 128), jnp.float32),
        pltpu.SemaphoreType.DMA,
        pltpu.SemaphoreType.DMA,
        pltpu.SemaphoreType.DMA,
        pltpu.SemaphoreType.DMA,
    ],
  )


def kernel(edge_index, num_nodes, spd_bias_weight):
    del num_nodes
    edge_index = edge_index.astype(jnp.int32)
    w = spd_bias_weight.astype(jnp.float32)

    adj = _adj_scatter()(edge_index)
    idx = _bfs()(adj.reshape(N * 16, 128))
    out4 = _hgather()(idx.reshape(-1), w.reshape(-1))
    return out4.reshape(N, 16, H, 128).transpose(0, 1, 3, 2).reshape(N, N, H)


# speedup vs baseline: 27.0879x; 1.0352x over previous
import functools

import jax
import jax.numpy as jnp
from jax import lax
from jax.experimental import pallas as pl
from jax.experimental.pallas import tpu as pltpu
from jax.experimental.pallas import tpu_sc as plsc

N = 2048
E = 32768
H = 8
MAX_D = 10
TBL = MAX_D + 2

_NC = 2
_NS = 16
_NT = _NC * _NS


@functools.cache
def _sc_mesh():
    return plsc.VectorSubcoreMesh(
        core_axis_name="c", subcore_axis_name="s",
        num_cores=_NC, num_subcores=_NS)


_WROWS = 32
_PASS = N // (_NT * _WROWS)
_ECHUNK = 8192


def _adj_body(edge_hbm, adj_hbm, win, srcv, dstv):
    c = lax.axis_index("c")
    s = lax.axis_index("s")
    tid = c * _NS + s
    ones16 = jnp.ones((16,), jnp.float32)

    for p in range(_PASS):
        rowbase = tid * (_WROWS * _PASS) + p * _WROWS

        def zro(i, _):
            win[pl.ds(i * 16, 16)] = jnp.zeros((16,), jnp.float32)
            return 0
        lax.fori_loop(0, _WROWS * N // 16, zro, 0)

        for ec in range(E // _ECHUNK):
            pltpu.sync_copy(edge_hbm.at[0, pl.ds(ec * _ECHUNK, _ECHUNK)], srcv)
            pltpu.sync_copy(edge_hbm.at[1, pl.ds(ec * _ECHUNK, _ECHUNK)], dstv)

            def grp(t, _):
                sv = srcv[pl.ds(t * 16, 16)]
                dv = dstv[pl.ds(t * 16, 16)]
                for a, b in ((sv, dv), (dv, sv)):
                    r = a - rowbase
                    ok = (r >= 0) & (r < _WROWS)
                    li = jnp.where(ok, r * N + b, 0)
                    plsc.store_scatter(win, [li], ones16, mask=ok)
                return 0
            lax.fori_loop(0, _ECHUNK // 16, grp, 0)

        off = pl.multiple_of(rowbase * N, _WROWS * N)
        pltpu.sync_copy(win, adj_hbm.at[pl.ds(off, _WROWS * N)])


@functools.cache
def _adj_scatter():
  return pl.kernel(
    _adj_body,
    out_type=jax.ShapeDtypeStruct((N * N,), jnp.float32),
    mesh=_sc_mesh(),
    compiler_params=pltpu.CompilerParams(use_tc_tiling_on_sc=False, needs_layout_passes=False),
    scratch_types=[
        pltpu.VMEM((_WROWS * N,), jnp.float32),
        pltpu.VMEM((_ECHUNK,), jnp.int32),
        pltpu.VMEM((_ECHUNK,), jnp.int32),
    ],
  )


_RB = 256


def _bfs_body(adj_ref, idxp_ref, adjb_ref):
    i = pl.program_id(0)
    row0 = i * _RB

    @pl.when(i == 0)
    def _():
        adjb_ref[...] = adj_ref[...].reshape(N, N).astype(jnp.bfloat16)

    adjb = adjb_ref[...]

    rows = lax.broadcasted_iota(jnp.int32, (_RB, N), 0) + row0
    cols = lax.broadcasted_iota(jnp.int32, (_RB, N), 1)
    reach0 = (rows == cols).astype(jnp.bfloat16)

    def cond(carry):
        d, done, _, _, _ = carry
        return jnp.logical_and(d <= MAX_D - 1, jnp.logical_not(done))

    def body(carry):
        d, _, prevcnt, reach, s_acc = carry
        nxt = jnp.dot(reach, adjb, preferred_element_type=jnp.float32) > 0.0
        newr = jnp.maximum(reach, nxt.astype(jnp.bfloat16))
        newr_f = newr.astype(jnp.float32)
        cnt = jnp.sum(newr_f)
        conv = cnt == prevcnt
        extra = jnp.where(conv, (MAX_D - 1 - d).astype(jnp.float32), 0.0)
        s_acc = s_acc + newr_f * (1.0 + extra)
        return d + 1, conv, cnt, newr, s_acc

    init = (jnp.int32(1), jnp.bool_(False), jnp.float32(_RB),
            reach0, reach0.astype(jnp.float32))
    _, _, _, _, s_acc = lax.while_loop(cond, body, init)

    idx = (MAX_D + 1.0) - s_acc
    idxp_ref[...] = idx.astype(jnp.int32).reshape(_RB * 16, 128)


@functools.cache
def _bfs():
  return pl.pallas_call(
    _bfs_body,
    grid=(N // _RB,),
    in_specs=[pl.BlockSpec((N * 16, 128), lambda i: (0, 0))],
    out_specs=pl.BlockSpec((_RB * 16, 128), lambda i: (i, 0)),
    out_shape=jax.ShapeDtypeStruct((N * 16, 128), jnp.int32),
    scratch_shapes=[pltpu.VMEM((N, N), jnp.bfloat16)],
  )


_IPT = N // _NT
_ICH = 2
_NCHUNK = _IPT // _ICH
_CIDX = _ICH * N
_CROW = _ICH * 16 * H


def _gather_body(idx_hbm, w_hbm, out_hbm, tblv, ib0, ib1, rb0, rb1,
                 semi0, semi1, semo0, semo1):
    c = lax.axis_index("c")
    s = lax.axis_index("s")
    tid = c * _NS + s
    pbase = tid * (_NCHUNK * _CIDX)
    obase = tid * (_NCHUNK * _CROW)

    pltpu.sync_copy(w_hbm, tblv)

    ibs = (ib0, ib1)
    rbs = (rb0, rb1)
    semis = (semi0, semi1)
    semos = (semo0, semo1)

    def idx_off(g):
        return pl.multiple_of(pbase + g * _CIDX, _CIDX)

    idma = [None, None]
    odma = [None, None]
    idma[0] = pltpu.async_copy(
        idx_hbm.at[pl.ds(idx_off(0), _CIDX)], ibs[0], semis[0])

    for g in range(_NCHUNK):
        cur = g % 2
        nxt = (g + 1) % 2
        if g + 1 < _NCHUNK:
            idma[nxt] = pltpu.async_copy(
                idx_hbm.at[pl.ds(idx_off(g + 1), _CIDX)], ibs[nxt], semis[nxt])
        idma[cur].wait()
        if odma[cur] is not None:
            odma[cur].wait()

        ib = ibs[cur]
        rb = rbs[cur]

        def block(b, _):
            iv = ib[pl.ds(b * 16, 16)]
            a0 = iv * H
            row0 = (b >> 7) * (16 * H) + ((b >> 3) & 15) * H
            lane0 = (b & 7) * 16
            for h in range(H):
                v = plsc.load_gather(tblv, [a0 + h])
                rb[row0 + h, pl.ds(lane0, 16)] = v
            return 0
        lax.fori_loop(0, _CIDX // 16, block, 0)

        odma[cur] = pltpu.async_copy(
            rb, out_hbm.at[pl.ds(pl.multiple_of(obase + g * _CROW, _CROW), _CROW)],
            semos[cur])

    odma[0].wait()
    odma[1].wait()


@functools.cache
def _hgather():
  return pl.kernel(
    _gather_body,
    out_type=jax.ShapeDtypeStruct((N * 16 * H, 128), jnp.float32),
    mesh=_sc_mesh(),
    compiler_params=pltpu.CompilerParams(use_tc_tiling_on_sc=False, needs_layout_passes=False),
    scratch_types=[
        pltpu.VMEM((TBL * H,), jnp.float32),
        pltpu.VMEM((_CIDX,), jnp.int32),
        pltpu.VMEM((_CIDX,), jnp.int32),
        pltpu.VMEM((_CROW, 128), jnp.float32),
        pltpu.VMEM((_CROW, 128), jnp.float32),
        pltpu.SemaphoreType.DMA,
        pltpu.SemaphoreType.DMA,
        pltpu.SemaphoreType.DMA,
        pltpu.SemaphoreType.DMA,
    ],
  )


def kernel(edge_index, num_nodes, spd_bias_weight):
    del num_nodes
    edge_index = edge_index.astype(jnp.int32)
    w = spd_bias_weight.astype(jnp.float32)

    adj = _adj_scatter()(edge_index)
    idx = _bfs()(adj.reshape(N * 16, 128))
    out4 = _hgather()(idx.reshape(-1), w.reshape(-1))
    return out4.reshape(N, 16, H, 128).transpose(0, 1, 3, 2).reshape(N, N, H)


# speedup vs baseline: 55.2185x; 2.0385x over previous
import functools

import jax
import jax.numpy as jnp
from jax import lax
from jax.experimental import pallas as pl
from jax.experimental.pallas import tpu as pltpu
from jax.experimental.pallas import tpu_sc as plsc

N = 2048
E = 32768
H = 8
MAX_D = 10
TBL = MAX_D + 2

_NC = 2
_NS = 16
_NT = _NC * _NS


@functools.cache
def _sc_mesh():
    return plsc.VectorSubcoreMesh(
        core_axis_name="c", subcore_axis_name="s",
        num_cores=_NC, num_subcores=_NS)


_WROWS = 32
_PASS = N // (_NT * _WROWS)
_ECHUNK = 8192


def _adj_body(edge_hbm, adj_hbm, win, srcv, dstv):
    c = lax.axis_index("c")
    s = lax.axis_index("s")
    tid = c * _NS + s
    ones16 = jnp.ones((16,), jnp.float32)

    for p in range(_PASS):
        rowbase = tid * (_WROWS * _PASS) + p * _WROWS

        @plsc.parallel_loop(0, _WROWS * N // 16, unroll=4)
        def zro(i):
            win[pl.ds(i * 16, 16)] = jnp.zeros((16,), jnp.float32)

        for ec in range(E // _ECHUNK):
            pltpu.sync_copy(edge_hbm.at[0, pl.ds(ec * _ECHUNK, _ECHUNK)], srcv)
            pltpu.sync_copy(edge_hbm.at[1, pl.ds(ec * _ECHUNK, _ECHUNK)], dstv)

            @plsc.parallel_loop(0, _ECHUNK // 16, unroll=2)
            def grp(t):
                sv = srcv[pl.ds(t * 16, 16)]
                dv = dstv[pl.ds(t * 16, 16)]
                for a, b in ((sv, dv), (dv, sv)):
                    r = a - rowbase
                    ok = (r >= 0) & (r < _WROWS)
                    li = jnp.where(ok, r * N + b, 0)
                    plsc.store_scatter(win, [li], ones16, mask=ok)

        off = pl.multiple_of(rowbase * N, _WROWS * N)
        pltpu.sync_copy(win, adj_hbm.at[pl.ds(off, _WROWS * N)])


@functools.cache
def _adj_scatter():
  return pl.kernel(
    _adj_body,
    out_type=jax.ShapeDtypeStruct((N * N,), jnp.float32),
    mesh=_sc_mesh(),
    compiler_params=pltpu.CompilerParams(use_tc_tiling_on_sc=False, needs_layout_passes=False),
    scratch_types=[
        pltpu.VMEM((_WROWS * N,), jnp.float32),
        pltpu.VMEM((_ECHUNK,), jnp.int32),
        pltpu.VMEM((_ECHUNK,), jnp.int32),
    ],
  )


_RB = 256


def _bfs_body(adj_ref, idxp_ref, adjb_ref):
    i = pl.program_id(0)
    row0 = i * _RB

    @pl.when(i == 0)
    def _():
        adjb_ref[...] = adj_ref[...].reshape(N, N).astype(jnp.bfloat16)

    adjb = adjb_ref[...]

    rows = lax.broadcasted_iota(jnp.int32, (_RB, N), 0) + row0
    cols = lax.broadcasted_iota(jnp.int32, (_RB, N), 1)
    reach0 = (rows == cols).astype(jnp.bfloat16)

    def cond(carry):
        d, done, _, _, _ = carry
        return jnp.logical_and(d <= MAX_D - 1, jnp.logical_not(done))

    def body(carry):
        d, _, prevcnt, reach, s_acc = carry
        nxt = jnp.dot(reach, adjb, preferred_element_type=jnp.float32) > 0.0
        newr = jnp.maximum(reach, nxt.astype(jnp.bfloat16))
        newr_f = newr.astype(jnp.float32)
        cnt = jnp.sum(newr_f)
        conv = cnt == prevcnt
        extra = jnp.where(conv, (MAX_D - 1 - d).astype(jnp.float32), 0.0)
        s_acc = s_acc + newr_f * (1.0 + extra)
        return d + 1, conv, cnt, newr, s_acc

    init = (jnp.int32(1), jnp.bool_(False), jnp.float32(_RB),
            reach0, reach0.astype(jnp.float32))
    _, _, _, _, s_acc = lax.while_loop(cond, body, init)

    idx = (MAX_D + 1.0) - s_acc
    idxp_ref[...] = idx.astype(jnp.int32).reshape(_RB * 16, 128)


@functools.cache
def _bfs():
  return pl.pallas_call(
    _bfs_body,
    grid=(N // _RB,),
    in_specs=[pl.BlockSpec((N * 16, 128), lambda i: (0, 0))],
    out_specs=pl.BlockSpec((_RB * 16, 128), lambda i: (i, 0)),
    out_shape=jax.ShapeDtypeStruct((N * 16, 128), jnp.int32),
    scratch_shapes=[pltpu.VMEM((N, N), jnp.bfloat16)],
  )


_IPT = N // _NT
_ICH = 2
_NCHUNK = _IPT // _ICH
_CIDX = _ICH * N
_CROW = _ICH * 16 * H


def _gather_body(idx_hbm, w_hbm, out_hbm, tblv, ib0, ib1, rb0, rb1,
                 semi0, semi1, semo0, semo1):
    c = lax.axis_index("c")
    s = lax.axis_index("s")
    tid = c * _NS + s
    pbase = tid * (_NCHUNK * _CIDX)
    obase = tid * (_NCHUNK * _CROW)

    pltpu.sync_copy(w_hbm, tblv)

    ibs = (ib0, ib1)
    rbs = (rb0, rb1)
    semis = (semi0, semi1)
    semos = (semo0, semo1)

    def idx_off(g):
        return pl.multiple_of(pbase + g * _CIDX, _CIDX)

    idma = [None, None]
    odma = [None, None]
    idma[0] = pltpu.async_copy(
        idx_hbm.at[pl.ds(idx_off(0), _CIDX)], ibs[0], semis[0])

    for g in range(_NCHUNK):
        cur = g % 2
        nxt = (g + 1) % 2
        if g + 1 < _NCHUNK:
            idma[nxt] = pltpu.async_copy(
                idx_hbm.at[pl.ds(idx_off(g + 1), _CIDX)], ibs[nxt], semis[nxt])
        idma[cur].wait()
        if odma[cur] is not None:
            odma[cur].wait()

        ib = ibs[cur]
        rb = rbs[cur]

        @plsc.parallel_loop(0, _CIDX // 16, unroll=2)
        def block(b):
            iv = ib[pl.ds(b * 16, 16)]
            a0 = iv * H
            row0 = (b >> 7) * (16 * H) + ((b >> 3) & 15) * H
            lane0 = (b & 7) * 16
            for h in range(H):
                v = plsc.load_gather(tblv, [a0 + h])
                rb[row0 + h, pl.ds(lane0, 16)] = v

        odma[cur] = pltpu.async_copy(
            rb, out_hbm.at[pl.ds(pl.multiple_of(obase + g * _CROW, _CROW), _CROW)],
            semos[cur])

    odma[0].wait()
    odma[1].wait()


@functools.cache
def _hgather():
  return pl.kernel(
    _gather_body,
    out_type=jax.ShapeDtypeStruct((N * 16 * H, 128), jnp.float32),
    mesh=_sc_mesh(),
    compiler_params=pltpu.CompilerParams(use_tc_tiling_on_sc=False, needs_layout_passes=False),
    scratch_types=[
        pltpu.VMEM((TBL * H,), jnp.float32),
        pltpu.VMEM((_CIDX,), jnp.int32),
        pltpu.VMEM((_CIDX,), jnp.int32),
        pltpu.VMEM((_CROW, 128), jnp.float32),
        pltpu.VMEM((_CROW, 128), jnp.float32),
        pltpu.SemaphoreType.DMA,
        pltpu.SemaphoreType.DMA,
        pltpu.SemaphoreType.DMA,
        pltpu.SemaphoreType.DMA,
    ],
  )


def kernel(edge_index, num_nodes, spd_bias_weight):
    del num_nodes
    edge_index = edge_index.astype(jnp.int32)
    w = spd_bias_weight.astype(jnp.float32)

    adj = _adj_scatter()(edge_index)
    idx = _bfs()(adj.reshape(N * 16, 128))
    out4 = _hgather()(idx.reshape(-1), w.reshape(-1))
    return out4.reshape(N, 16, H, 128).transpose(0, 1, 3, 2).reshape(N, N, H)


# speedup vs baseline: 63.2650x; 1.1457x over previous
import functools

import jax
import jax.numpy as jnp
from jax import lax
from jax.experimental import pallas as pl
from jax.experimental.pallas import tpu as pltpu
from jax.experimental.pallas import tpu_sc as plsc

N = 2048
E = 32768
H = 8
MAX_D = 10
TBL = MAX_D + 2

_NC = 2
_NS = 16
_NT = _NC * _NS


@functools.cache
def _sc_mesh():
    return plsc.VectorSubcoreMesh(
        core_axis_name="c", subcore_axis_name="s",
        num_cores=_NC, num_subcores=_NS)


_WROWS = 32
_PASS = N // (_NT * _WROWS)
_ECHUNK = 8192


def _adj_body(edge_hbm, adj_hbm, win, srcv, dstv):
    c = lax.axis_index("c")
    s = lax.axis_index("s")
    tid = c * _NS + s
    ones16 = jnp.ones((16,), jnp.float32)

    for p in range(_PASS):
        rowbase = tid * (_WROWS * _PASS) + p * _WROWS

        @plsc.parallel_loop(0, _WROWS * N // 16, unroll=4)
        def zro(i):
            win[pl.ds(i * 16, 16)] = jnp.zeros((16,), jnp.float32)

        for ec in range(E // _ECHUNK):
            pltpu.sync_copy(edge_hbm.at[0, pl.ds(ec * _ECHUNK, _ECHUNK)], srcv)
            pltpu.sync_copy(edge_hbm.at[1, pl.ds(ec * _ECHUNK, _ECHUNK)], dstv)

            @plsc.parallel_loop(0, _ECHUNK // 16, unroll=2)
            def grp(t):
                sv = srcv[pl.ds(t * 16, 16)]
                dv = dstv[pl.ds(t * 16, 16)]
                for a, b in ((sv, dv), (dv, sv)):
                    r = a - rowbase
                    ok = (r >= 0) & (r < _WROWS)
                    li = jnp.where(ok, r * N + b, 0)
                    plsc.store_scatter(win, [li], ones16, mask=ok)

        off = pl.multiple_of(rowbase * N, _WROWS * N)
        pltpu.sync_copy(win, adj_hbm.at[pl.ds(off, _WROWS * N)])


@functools.cache
def _adj_scatter():
  return pl.kernel(
    _adj_body,
    out_type=jax.ShapeDtypeStruct((N * N,), jnp.float32),
    mesh=_sc_mesh(),
    compiler_params=pltpu.CompilerParams(use_tc_tiling_on_sc=False, needs_layout_passes=False),
    scratch_types=[
        pltpu.VMEM((_WROWS * N,), jnp.float32),
        pltpu.VMEM((_ECHUNK,), jnp.int32),
        pltpu.VMEM((_ECHUNK,), jnp.int32),
    ],
  )


_RB = 256


def _bfs_body(adj_ref, idxp_ref, adjb_ref):
    i = pl.program_id(0)
    row0 = i * _RB

    @pl.when(i == 0)
    def _():
        adjb_ref[...] = adj_ref[...].reshape(N, N).astype(jnp.bfloat16)

    adjb = adjb_ref[...]
    onesb = jnp.ones((8, _RB), jnp.bfloat16)

    def count(x_bf16):
        return jnp.sum(jnp.dot(onesb, x_bf16, preferred_element_type=jnp.float32))

    rows = lax.broadcasted_iota(jnp.int32, (_RB, N), 0) + row0
    cols = lax.broadcasted_iota(jnp.int32, (_RB, N), 1)
    reach0 = (rows == cols).astype(jnp.bfloat16)
    reach1 = jnp.maximum(reach0, adjb_ref[pl.ds(row0, _RB), :])

    def cond(carry):
        d, done, _, _, _ = carry
        return jnp.logical_and(d <= MAX_D - 1, jnp.logical_not(done))

    def body(carry):
        d, _, prevcnt, reach, s_acc = carry
        nxt = jnp.dot(reach, adjb, preferred_element_type=jnp.float32) > 0.0
        newr = jnp.maximum(reach, nxt.astype(jnp.bfloat16))
        cnt = count(newr)
        conv = cnt == prevcnt
        extra = jnp.where(conv, (MAX_D - 1 - d).astype(jnp.float32), 0.0)
        s_acc = s_acc + newr.astype(jnp.float32) * (1.0 + extra)
        return d + 1, conv, cnt, newr, s_acc

    init = (jnp.int32(2), jnp.bool_(False), count(reach1),
            reach1, reach0.astype(jnp.float32) + reach1.astype(jnp.float32))
    _, _, _, _, s_acc = lax.while_loop(cond, body, init)

    idx = (MAX_D + 1.0) - s_acc
    idxp_ref[...] = idx.astype(jnp.int32).reshape(_RB * 16, 128)


@functools.cache
def _bfs():
  return pl.pallas_call(
    _bfs_body,
    grid=(N // _RB,),
    in_specs=[pl.BlockSpec((N * 16, 128), lambda i: (0, 0))],
    out_specs=pl.BlockSpec((_RB * 16, 128), lambda i: (i, 0)),
    out_shape=jax.ShapeDtypeStruct((N * 16, 128), jnp.int32),
    scratch_shapes=[pltpu.VMEM((N, N), jnp.bfloat16)],
  )


_IPT = N // _NT
_ICH = 2
_NCHUNK = _IPT // _ICH
_CIDX = _ICH * N
_CROW = _ICH * 16 * H


def _gather_body(idx_hbm, w_hbm, out_hbm, tblv, ib0, ib1, rb0, rb1,
                 semi0, semi1, semo0, semo1):
    c = lax.axis_index("c")
    s = lax.axis_index("s")
    tid = c * _NS + s
    pbase = tid * (_NCHUNK * _CIDX)
    obase = tid * (_NCHUNK * _CROW)

    pltpu.sync_copy(w_hbm, tblv)

    ibs = (ib0, ib1)
    rbs = (rb0, rb1)
    semis = (semi0, semi1)
    semos = (semo0, semo1)

    def idx_off(g):
        return pl.multiple_of(pbase + g * _CIDX, _CIDX)

    idma = [None, None]
    odma = [None, None]
    idma[0] = pltpu.async_copy(
        idx_hbm.at[pl.ds(idx_off(0), _CIDX)], ibs[0], semis[0])

    for g in range(_NCHUNK):
        cur = g % 2
        nxt = (g + 1) % 2
        if g + 1 < _NCHUNK:
            idma[nxt] = pltpu.async_copy(
                idx_hbm.at[pl.ds(idx_off(g + 1), _CIDX)], ibs[nxt], semis[nxt])
        idma[cur].wait()
        if odma[cur] is not None:
            odma[cur].wait()

        ib = ibs[cur]
        rb = rbs[cur]

        @plsc.parallel_loop(0, _CIDX // 16, unroll=2)
        def block(b):
            iv = ib[pl.ds(b * 16, 16)]
            a0 = iv * H
            row0 = (b >> 7) * (16 * H) + ((b >> 3) & 15) * H
            lane0 = (b & 7) * 16
            for h in range(H):
                v = plsc.load_gather(tblv, [a0 + h])
                rb[row0 + h, pl.ds(lane0, 16)] = v

        odma[cur] = pltpu.async_copy(
            rb, out_hbm.at[pl.ds(pl.multiple_of(obase + g * _CROW, _CROW), _CROW)],
            semos[cur])

    odma[0].wait()
    odma[1].wait()


@functools.cache
def _hgather():
  return pl.kernel(
    _gather_body,
    out_type=jax.ShapeDtypeStruct((N * 16 * H, 128), jnp.float32),
    mesh=_sc_mesh(),
    compiler_params=pltpu.CompilerParams(use_tc_tiling_on_sc=False, needs_layout_passes=False),
    scratch_types=[
        pltpu.VMEM((TBL * H,), jnp.float32),
        pltpu.VMEM((_CIDX,), jnp.int32),
        pltpu.VMEM((_CIDX,), jnp.int32),
        pltpu.VMEM((_CROW, 128), jnp.float32),
        pltpu.VMEM((_CROW, 128), jnp.float32),
        pltpu.SemaphoreType.DMA,
        pltpu.SemaphoreType.DMA,
        pltpu.SemaphoreType.DMA,
        pltpu.SemaphoreType.DMA,
    ],
  )


def kernel(edge_index, num_nodes, spd_bias_weight):
    del num_nodes
    edge_index = edge_index.astype(jnp.int32)
    w = spd_bias_weight.astype(jnp.float32)

    adj = _adj_scatter()(edge_index)
    idx = _bfs()(adj.reshape(N * 16, 128))
    out4 = _hgather()(idx.reshape(-1), w.reshape(-1))
    return out4.reshape(N, 16, H, 128).transpose(0, 1, 3, 2).reshape(N, N, H)


# speedup vs baseline: 63.5520x; 1.0045x over previous
import functools

import jax
import jax.numpy as jnp
from jax import lax
from jax.experimental import pallas as pl
from jax.experimental.pallas import tpu as pltpu
from jax.experimental.pallas import tpu_sc as plsc

N = 2048
E = 32768
H = 8
MAX_D = 10
TBL = MAX_D + 2

_NC = 2
_NS = 16
_NT = _NC * _NS


@functools.cache
def _sc_mesh():
    return plsc.VectorSubcoreMesh(
        core_axis_name="c", subcore_axis_name="s",
        num_cores=_NC, num_subcores=_NS)


_WROWS = 32
_PASS = N // (_NT * _WROWS)
_ECHUNK = 8192


def _adj_body(edge_hbm, adj_hbm, win, srcv, dstv):
    c = lax.axis_index("c")
    s = lax.axis_index("s")
    tid = c * _NS + s
    ones16 = jnp.ones((16,), jnp.float32)

    for p in range(_PASS):
        rowbase = tid * (_WROWS * _PASS) + p * _WROWS

        @plsc.parallel_loop(0, _WROWS * N // 16, unroll=4)
        def zro(i):
            win[pl.ds(i * 16, 16)] = jnp.zeros((16,), jnp.float32)

        for ec in range(E // _ECHUNK):
            pltpu.sync_copy(edge_hbm.at[0, pl.ds(ec * _ECHUNK, _ECHUNK)], srcv)
            pltpu.sync_copy(edge_hbm.at[1, pl.ds(ec * _ECHUNK, _ECHUNK)], dstv)

            @plsc.parallel_loop(0, _ECHUNK // 16, unroll=2)
            def grp(t):
                sv = srcv[pl.ds(t * 16, 16)]
                dv = dstv[pl.ds(t * 16, 16)]
                for a, b in ((sv, dv), (dv, sv)):
                    r = a - rowbase
                    ok = (r >= 0) & (r < _WROWS)
                    li = jnp.where(ok, r * N + b, 0)
                    plsc.store_scatter(win, [li], ones16, mask=ok)

        off = pl.multiple_of(rowbase * N, _WROWS * N)
        pltpu.sync_copy(win, adj_hbm.at[pl.ds(off, _WROWS * N)])


@functools.cache
def _adj_scatter():
  return pl.kernel(
    _adj_body,
    out_type=jax.ShapeDtypeStruct((N * N,), jnp.float32),
    mesh=_sc_mesh(),
    compiler_params=pltpu.CompilerParams(use_tc_tiling_on_sc=False, needs_layout_passes=False),
    scratch_types=[
        pltpu.VMEM((_WROWS * N,), jnp.float32),
        pltpu.VMEM((_ECHUNK,), jnp.int32),
        pltpu.VMEM((_ECHUNK,), jnp.int32),
    ],
  )


_RB = 256


def _bfs_body(adj_ref, idxp_ref, adjb_ref):
    i = pl.program_id(0)
    row0 = i * _RB

    @pl.when(i == 0)
    def _():
        adjb_ref[...] = adj_ref[...].reshape(N, N).astype(jnp.int8)

    adjb = adjb_ref[...]
    onesb = jnp.ones((8, _RB), jnp.int8)

    def count(x_s8):
        return jnp.sum(jnp.dot(onesb, x_s8, preferred_element_type=jnp.int32))

    rows = lax.broadcasted_iota(jnp.int32, (_RB, N), 0) + row0
    cols = lax.broadcasted_iota(jnp.int32, (_RB, N), 1)
    reach0 = (rows == cols).astype(jnp.int8)
    reach1 = reach0 | adjb_ref[pl.ds(row0, _RB), :]

    def cond(carry):
        d, done, _, _, _ = carry
        return jnp.logical_and(d <= MAX_D - 1, jnp.logical_not(done))

    def body(carry):
        d, _, prevcnt, reach, s_acc = carry
        nxt = jnp.dot(reach, adjb, preferred_element_type=jnp.int32) > 0
        newr = reach | nxt.astype(jnp.int8)
        cnt = count(newr)
        conv = cnt == prevcnt
        extra = jnp.where(conv, (MAX_D - 1 - d).astype(jnp.float32), 0.0)
        s_acc = s_acc + newr.astype(jnp.float32) * (1.0 + extra)
        return d + 1, conv, cnt, newr, s_acc

    init = (jnp.int32(2), jnp.bool_(False), count(reach1),
            reach1, reach0.astype(jnp.float32) + reach1.astype(jnp.float32))
    _, _, _, _, s_acc = lax.while_loop(cond, body, init)

    idx = (MAX_D + 1.0) - s_acc
    idxp_ref[...] = idx.astype(jnp.int32).reshape(_RB * 16, 128)


@functools.cache
def _bfs():
  return pl.pallas_call(
    _bfs_body,
    grid=(N // _RB,),
    in_specs=[pl.BlockSpec((N * 16, 128), lambda i: (0, 0))],
    out_specs=pl.BlockSpec((_RB * 16, 128), lambda i: (i, 0)),
    out_shape=jax.ShapeDtypeStruct((N * 16, 128), jnp.int32),
    scratch_shapes=[pltpu.VMEM((N, N), jnp.int8)],
  )


_IPT = N // _NT
_ICH = 2
_NCHUNK = _IPT // _ICH
_CIDX = _ICH * N
_CROW = _ICH * 16 * H


def _gather_body(idx_hbm, w_hbm, out_hbm, tblv, ib0, ib1, rb0, rb1,
                 semi0, semi1, semo0, semo1):
    c = lax.axis_index("c")
    s = lax.axis_index("s")
    tid = c * _NS + s
    pbase = tid * (_NCHUNK * _CIDX)
    obase = tid * (_NCHUNK * _CROW)

    pltpu.sync_copy(w_hbm, tblv)

    ibs = (ib0, ib1)
    rbs = (rb0, rb1)
    semis = (semi0, semi1)
    semos = (semo0, semo1)

    def idx_off(g):
        return pl.multiple_of(pbase + g * _CIDX, _CIDX)

    idma = [None, None]
    odma = [None, None]
    idma[0] = pltpu.async_copy(
        idx_hbm.at[pl.ds(idx_off(0), _CIDX)], ibs[0], semis[0])

    for g in range(_NCHUNK):
        cur = g % 2
        nxt = (g + 1) % 2
        if g + 1 < _NCHUNK:
            idma[nxt] = pltpu.async_copy(
                idx_hbm.at[pl.ds(idx_off(g + 1), _CIDX)], ibs[nxt], semis[nxt])
        idma[cur].wait()
        if odma[cur] is not None:
            odma[cur].wait()

        ib = ibs[cur]
        rb = rbs[cur]

        @plsc.parallel_loop(0, _CIDX // 16, unroll=2)
        def block(b):
            iv = ib[pl.ds(b * 16, 16)]
            a0 = iv * H
            row0 = (b >> 7) * (16 * H) + ((b >> 3) & 15) * H
            lane0 = (b & 7) * 16
            for h in range(H):
                v = plsc.load_gather(tblv, [a0 + h])
                rb[row0 + h, pl.ds(lane0, 16)] = v

        odma[cur] = pltpu.async_copy(
            rb, out_hbm.at[pl.ds(pl.multiple_of(obase + g * _CROW, _CROW), _CROW)],
            semos[cur])

    odma[0].wait()
    odma[1].wait()


@functools.cache
def _hgather():
  return pl.kernel(
    _gather_body,
    out_type=jax.ShapeDtypeStruct((N * 16 * H, 128), jnp.float32),
    mesh=_sc_mesh(),
    compiler_params=pltpu.CompilerParams(use_tc_tiling_on_sc=False, needs_layout_passes=False),
    scratch_types=[
        pltpu.VMEM((TBL * H,), jnp.float32),
        pltpu.VMEM((_CIDX,), jnp.int32),
        pltpu.VMEM((_CIDX,), jnp.int32),
        pltpu.VMEM((_CROW, 128), jnp.float32),
        pltpu.VMEM((_CROW, 128), jnp.float32),
        pltpu.SemaphoreType.DMA,
        pltpu.SemaphoreType.DMA,
        pltpu.SemaphoreType.DMA,
        pltpu.SemaphoreType.DMA,
    ],
  )


def kernel(edge_index, num_nodes, spd_bias_weight):
    del num_nodes
    edge_index = edge_index.astype(jnp.int32)
    w = spd_bias_weight.astype(jnp.float32)

    adj = _adj_scatter()(edge_index)
    idx = _bfs()(adj.reshape(N * 16, 128))
    out4 = _hgather()(idx.reshape(-1), w.reshape(-1))
    return out4.reshape(N, 16, H, 128).transpose(0, 1, 3, 2).reshape(N, N, H)


# speedup vs baseline: 70.8621x; 1.1150x over previous
import functools

import jax
import jax.numpy as jnp
from jax import lax
from jax.experimental import pallas as pl
from jax.experimental.pallas import tpu as pltpu
from jax.experimental.pallas import tpu_sc as plsc

N = 2048
E = 32768
H = 8
MAX_D = 10
TBL = MAX_D + 2

_NC = 2
_NS = 16
_NT = _NC * _NS


@functools.cache
def _sc_mesh():
    return plsc.VectorSubcoreMesh(
        core_axis_name="c", subcore_axis_name="s",
        num_cores=_NC, num_subcores=_NS)


_WROWS = 32
_PASS = N // (_NT * _WROWS)
_ECHUNK = 8192


def _adj_body(edge_hbm, adj_hbm, win, srcv, dstv):
    c = lax.axis_index("c")
    s = lax.axis_index("s")
    tid = c * _NS + s
    ones16 = jnp.ones((16,), jnp.float32)

    for p in range(_PASS):
        rowbase = tid * (_WROWS * _PASS) + p * _WROWS

        @plsc.parallel_loop(0, _WROWS * N // 16, unroll=4)
        def zro(i):
            win[pl.ds(i * 16, 16)] = jnp.zeros((16,), jnp.float32)

        for ec in range(E // _ECHUNK):
            pltpu.sync_copy(edge_hbm.at[0, pl.ds(ec * _ECHUNK, _ECHUNK)], srcv)
            pltpu.sync_copy(edge_hbm.at[1, pl.ds(ec * _ECHUNK, _ECHUNK)], dstv)

            @plsc.parallel_loop(0, _ECHUNK // 16, unroll=2)
            def grp(t):
                sv = srcv[pl.ds(t * 16, 16)]
                dv = dstv[pl.ds(t * 16, 16)]
                for a, b in ((sv, dv), (dv, sv)):
                    r = a - rowbase
                    ok = (r >= 0) & (r < _WROWS)
                    li = jnp.where(ok, r * N + b, 0)
                    plsc.store_scatter(win, [li], ones16, mask=ok)

        off = pl.multiple_of(rowbase * N, _WROWS * N)
        pltpu.sync_copy(win, adj_hbm.at[pl.ds(off, _WROWS * N)])


@functools.cache
def _adj_scatter():
  return pl.kernel(
    _adj_body,
    out_type=jax.ShapeDtypeStruct((N * N,), jnp.float32),
    mesh=_sc_mesh(),
    compiler_params=pltpu.CompilerParams(use_tc_tiling_on_sc=False, needs_layout_passes=False),
    scratch_types=[
        pltpu.VMEM((_WROWS * N,), jnp.float32),
        pltpu.VMEM((_ECHUNK,), jnp.int32),
        pltpu.VMEM((_ECHUNK,), jnp.int32),
    ],
  )


_RB = 256


def _bfs_body(adj_ref, idxp_ref, adjb_ref):
    i = pl.program_id(0)
    row0 = i * _RB

    @pl.when(i == 0)
    def _():
        adjb_ref[...] = adj_ref[...].reshape(N, N).astype(jnp.int8)

    adjb = adjb_ref[...]
    onesb = jnp.ones((8, _RB), jnp.int8)

    def count(x_s8):
        return jnp.sum(jnp.dot(onesb, x_s8, preferred_element_type=jnp.int32))

    rows = lax.broadcasted_iota(jnp.int32, (_RB, N), 0) + row0
    cols = lax.broadcasted_iota(jnp.int32, (_RB, N), 1)
    reach0 = (rows == cols).astype(jnp.int8)
    reach1 = reach0 | adjb_ref[pl.ds(row0, _RB), :]

    def cond(carry):
        d, done, _, _, _ = carry
        return jnp.logical_and(d <= MAX_D - 1, jnp.logical_not(done))

    def body(carry):
        d, _, prevcnt, reach, s_acc = carry
        nxt = jnp.dot(reach, adjb, preferred_element_type=jnp.int32) > 0
        newr = reach | nxt.astype(jnp.int8)
        cnt = count(newr)
        conv = jnp.logical_or(cnt == prevcnt, cnt == 8 * _RB * N)
        extra = jnp.where(conv, (MAX_D - 1 - d).astype(jnp.float32), 0.0)
        s_acc = s_acc + newr.astype(jnp.float32) * (1.0 + extra)
        return d + 1, conv, cnt, newr, s_acc

    init = (jnp.int32(2), jnp.bool_(False), count(reach1),
            reach1, reach0.astype(jnp.float32) + reach1.astype(jnp.float32))
    _, _, _, _, s_acc = lax.while_loop(cond, body, init)

    idx = (MAX_D + 1.0) - s_acc
    idxp_ref[...] = idx.astype(jnp.int32).reshape(_RB * 16, 128)


@functools.cache
def _bfs():
  return pl.pallas_call(
    _bfs_body,
    grid=(N // _RB,),
    in_specs=[pl.BlockSpec((N * 16, 128), lambda i: (0, 0))],
    out_specs=pl.BlockSpec((_RB * 16, 128), lambda i: (i, 0)),
    out_shape=jax.ShapeDtypeStruct((N * 16, 128), jnp.int32),
    scratch_shapes=[pltpu.VMEM((N, N), jnp.int8)],
  )


_IPT = N // _NT
_ICH = 2
_NCHUNK = _IPT // _ICH
_CIDX = _ICH * N
_CROW = _ICH * 16 * H


def _gather_body(idx_hbm, w_hbm, out_hbm, tblv, ib0, ib1, rb0, rb1,
                 semi0, semi1, semo0, semo1):
    c = lax.axis_index("c")
    s = lax.axis_index("s")
    tid = c * _NS + s
    pbase = tid * (_NCHUNK * _CIDX)
    obase = tid * (_NCHUNK * _CROW)

    pltpu.sync_copy(w_hbm, tblv)

    ibs = (ib0, ib1)
    rbs = (rb0, rb1)
    semis = (semi0, semi1)
    semos = (semo0, semo1)

    def idx_off(g):
        return pl.multiple_of(pbase + g * _CIDX, _CIDX)

    idma = [None, None]
    odma = [None, None]
    idma[0] = pltpu.async_copy(
        idx_hbm.at[pl.ds(idx_off(0), _CIDX)], ibs[0], semis[0])

    for g in range(_NCHUNK):
        cur = g % 2
        nxt = (g + 1) % 2
        if g + 1 < _NCHUNK:
            idma[nxt] = pltpu.async_copy(
                idx_hbm.at[pl.ds(idx_off(g + 1), _CIDX)], ibs[nxt], semis[nxt])
        idma[cur].wait()
        if odma[cur] is not None:
            odma[cur].wait()

        ib = ibs[cur]
        rb = rbs[cur]

        @plsc.parallel_loop(0, _CIDX // 16, unroll=2)
        def block(b):
            iv = ib[pl.ds(b * 16, 16)]
            a0 = iv * H
            row0 = (b >> 7) * (16 * H) + ((b >> 3) & 15) * H
            lane0 = (b & 7) * 16
            for h in range(H):
                v = plsc.load_gather(tblv, [a0 + h])
                rb[row0 + h, pl.ds(lane0, 16)] = v

        odma[cur] = pltpu.async_copy(
            rb, out_hbm.at[pl.ds(pl.multiple_of(obase + g * _CROW, _CROW), _CROW)],
            semos[cur])

    odma[0].wait()
    odma[1].wait()


@functools.cache
def _hgather():
  return pl.kernel(
    _gather_body,
    out_type=jax.ShapeDtypeStruct((N * 16 * H, 128), jnp.float32),
    mesh=_sc_mesh(),
    compiler_params=pltpu.CompilerParams(use_tc_tiling_on_sc=False, needs_layout_passes=False),
    scratch_types=[
        pltpu.VMEM((TBL * H,), jnp.float32),
        pltpu.VMEM((_CIDX,), jnp.int32),
        pltpu.VMEM((_CIDX,), jnp.int32),
        pltpu.VMEM((_CROW, 128), jnp.float32),
        pltpu.VMEM((_CROW, 128), jnp.float32),
        pltpu.SemaphoreType.DMA,
        pltpu.SemaphoreType.DMA,
        pltpu.SemaphoreType.DMA,
        pltpu.SemaphoreType.DMA,
    ],
  )


def kernel(edge_index, num_nodes, spd_bias_weight):
    del num_nodes
    edge_index = edge_index.astype(jnp.int32)
    w = spd_bias_weight.astype(jnp.float32)

    adj = _adj_scatter()(edge_index)
    idx = _bfs()(adj.reshape(N * 16, 128))
    out4 = _hgather()(idx.reshape(-1), w.reshape(-1))
    return out4.reshape(N, 16, H, 128).transpose(0, 1, 3, 2).reshape(N, N, H)


# speedup vs baseline: 72.4018x; 1.0217x over previous
import functools

import jax
import jax.numpy as jnp
from jax import lax
from jax.experimental import pallas as pl
from jax.experimental.pallas import tpu as pltpu
from jax.experimental.pallas import tpu_sc as plsc

N = 2048
E = 32768
H = 8
MAX_D = 10
TBL = MAX_D + 2

_NC = 2
_NS = 16
_NT = _NC * _NS


@functools.cache
def _sc_mesh():
    return plsc.VectorSubcoreMesh(
        core_axis_name="c", subcore_axis_name="s",
        num_cores=_NC, num_subcores=_NS)


_WROWS = 32
_PASS = N // (_NT * _WROWS)
_ECHUNK = 8192


def _adj_body(edge_hbm, adj_hbm, win, srcv, dstv):
    c = lax.axis_index("c")
    s = lax.axis_index("s")
    tid = c * _NS + s
    ones16 = jnp.ones((16,), jnp.float32)

    for p in range(_PASS):
        rowbase = tid * (_WROWS * _PASS) + p * _WROWS

        @plsc.parallel_loop(0, _WROWS * N // 16, unroll=4)
        def zro(i):
            win[pl.ds(i * 16, 16)] = jnp.zeros((16,), jnp.float32)

        for ec in range(E // _ECHUNK):
            pltpu.sync_copy(edge_hbm.at[0, pl.ds(ec * _ECHUNK, _ECHUNK)], srcv)
            pltpu.sync_copy(edge_hbm.at[1, pl.ds(ec * _ECHUNK, _ECHUNK)], dstv)

            @plsc.parallel_loop(0, _ECHUNK // 16, unroll=2)
            def grp(t):
                sv = srcv[pl.ds(t * 16, 16)]
                dv = dstv[pl.ds(t * 16, 16)]
                for a, b in ((sv, dv), (dv, sv)):
                    r = a - rowbase
                    ok = (r >= 0) & (r < _WROWS)
                    li = jnp.where(ok, r * N + b, 0)
                    plsc.store_scatter(win, [li], ones16, mask=ok)

        off = pl.multiple_of(rowbase * N, _WROWS * N)
        pltpu.sync_copy(win, adj_hbm.at[pl.ds(off, _WROWS * N)])


@functools.cache
def _adj_scatter():
  return pl.kernel(
    _adj_body,
    out_type=jax.ShapeDtypeStruct((N * N,), jnp.float32),
    mesh=_sc_mesh(),
    compiler_params=pltpu.CompilerParams(use_tc_tiling_on_sc=False, needs_layout_passes=False),
    scratch_types=[
        pltpu.VMEM((_WROWS * N,), jnp.float32),
        pltpu.VMEM((_ECHUNK,), jnp.int32),
        pltpu.VMEM((_ECHUNK,), jnp.int32),
    ],
  )


_RB = 512


def _bfs_body(adj_ref, idxp_ref, adjb_ref):
    i = pl.program_id(0)
    row0 = i * _RB

    @pl.when(i == 0)
    def _():
        adjb_ref[...] = adj_ref[...].reshape(N, N).astype(jnp.int8)

    adjb = adjb_ref[...]
    onesb = jnp.ones((8, _RB), jnp.int8)

    def count(x_s8):
        return jnp.sum(jnp.dot(onesb, x_s8, preferred_element_type=jnp.int32))

    rows = lax.broadcasted_iota(jnp.int32, (_RB, N), 0) + row0
    cols = lax.broadcasted_iota(jnp.int32, (_RB, N), 1)
    reach0 = (rows == cols).astype(jnp.int8)
    reach1 = reach0 | adjb_ref[pl.ds(row0, _RB), :]

    def cond(carry):
        d, done, _, _, _ = carry
        return jnp.logical_and(d <= MAX_D - 1, jnp.logical_not(done))

    def body(carry):
        d, _, prevcnt, reach, s_acc = carry
        nxt = jnp.dot(reach, adjb, preferred_element_type=jnp.int32) > 0
        newr = reach | nxt.astype(jnp.int8)
        cnt = count(newr)
        conv = jnp.logical_or(cnt == prevcnt, cnt == 8 * _RB * N)
        extra = jnp.where(conv, (MAX_D - 1 - d).astype(jnp.float32), 0.0)
        s_acc = s_acc + newr.astype(jnp.float32) * (1.0 + extra)
        return d + 1, conv, cnt, newr, s_acc

    init = (jnp.int32(2), jnp.bool_(False), count(reach1),
            reach1, reach0.astype(jnp.float32) + reach1.astype(jnp.float32))
    _, _, _, _, s_acc = lax.while_loop(cond, body, init)

    idx = (MAX_D + 1.0) - s_acc
    idxp_ref[...] = idx.astype(jnp.int32).reshape(_RB * 16, 128)


@functools.cache
def _bfs():
  return pl.pallas_call(
    _bfs_body,
    grid=(N // _RB,),
    in_specs=[pl.BlockSpec((N * 16, 128), lambda i: (0, 0))],
    out_specs=pl.BlockSpec((_RB * 16, 128), lambda i: (i, 0)),
    out_shape=jax.ShapeDtypeStruct((N * 16, 128), jnp.int32),
    scratch_shapes=[pltpu.VMEM((N, N), jnp.int8)],
  )


_IPT = N // _NT
_ICH = 2
_NCHUNK = _IPT // _ICH
_CIDX = _ICH * N
_CROW = _ICH * 16 * H


def _gather_body(idx_hbm, w_hbm, out_hbm, tblv, ib0, ib1, rb0, rb1,
                 semi0, semi1, semo0, semo1):
    c = lax.axis_index("c")
    s = lax.axis_index("s")
    tid = c * _NS + s
    pbase = tid * (_NCHUNK * _CIDX)
    obase = tid * (_NCHUNK * _CROW)

    pltpu.sync_copy(w_hbm, tblv)

    ibs = (ib0, ib1)
    rbs = (rb0, rb1)
    semis = (semi0, semi1)
    semos = (semo0, semo1)

    def idx_off(g):
        return pl.multiple_of(pbase + g * _CIDX, _CIDX)

    idma = [None, None]
    odma = [None, None]
    idma[0] = pltpu.async_copy(
        idx_hbm.at[pl.ds(idx_off(0), _CIDX)], ibs[0], semis[0])

    for g in range(_NCHUNK):
        cur = g % 2
        nxt = (g + 1) % 2
        if g + 1 < _NCHUNK:
            idma[nxt] = pltpu.async_copy(
                idx_hbm.at[pl.ds(idx_off(g + 1), _CIDX)], ibs[nxt], semis[nxt])
        idma[cur].wait()
        if odma[cur] is not None:
            odma[cur].wait()

        ib = ibs[cur]
        rb = rbs[cur]

        @plsc.parallel_loop(0, _CIDX // 16, unroll=2)
        def block(b):
            iv = ib[pl.ds(b * 16, 16)]
            a0 = iv * H
            row0 = (b >> 7) * (16 * H) + ((b >> 3) & 15) * H
            lane0 = (b & 7) * 16
            for h in range(H):
                v = plsc.load_gather(tblv, [a0 + h])
                rb[row0 + h, pl.ds(lane0, 16)] = v

        odma[cur] = pltpu.async_copy(
            rb, out_hbm.at[pl.ds(pl.multiple_of(obase + g * _CROW, _CROW), _CROW)],
            semos[cur])

    odma[0].wait()
    odma[1].wait()


@functools.cache
def _hgather():
  return pl.kernel(
    _gather_body,
    out_type=jax.ShapeDtypeStruct((N * 16 * H, 128), jnp.float32),
    mesh=_sc_mesh(),
    compiler_params=pltpu.CompilerParams(use_tc_tiling_on_sc=False, needs_layout_passes=False),
    scratch_types=[
        pltpu.VMEM((TBL * H,), jnp.float32),
        pltpu.VMEM((_CIDX,), jnp.int32),
        pltpu.VMEM((_CIDX,), jnp.int32),
        pltpu.VMEM((_CROW, 128), jnp.float32),
        pltpu.VMEM((_CROW, 128), jnp.float32),
        pltpu.SemaphoreType.DMA,
        pltpu.SemaphoreType.DMA,
        pltpu.SemaphoreType.DMA,
        pltpu.SemaphoreType.DMA,
    ],
  )


def kernel(edge_index, num_nodes, spd_bias_weight):
    del num_nodes
    edge_index = edge_index.astype(jnp.int32)
    w = spd_bias_weight.astype(jnp.float32)

    adj = _adj_scatter()(edge_index)
    idx = _bfs()(adj.reshape(N * 16, 128))
    out4 = _hgather()(idx.reshape(-1), w.reshape(-1))
    return out4.reshape(N, 16, H, 128).transpose(0, 1, 3, 2).reshape(N, N, H)
